# baseline (speedup 1.0000x reference)
import functools

import jax
import jax.numpy as jnp
from jax.experimental import pallas as pl
from jax.experimental.pallas import tpu as pltpu


def _layer_body(adj_ref, xin_full_ref, xin_blk_ref, w_ref, out_ref, *, nfeat):
    support = jnp.dot(
        adj_ref[...], xin_full_ref[...], preferred_element_type=jnp.float32
    )
    h = jnp.dot(
        xin_blk_ref[...], w_ref[:nfeat, :], preferred_element_type=jnp.float32
    ) + jnp.dot(support, w_ref[nfeat:, :], preferred_element_type=jnp.float32)
    out_ref[...] = jnp.maximum(h, 0.0)


def _layer(xin, adj, w, block_m):
    n, nfeat = xin.shape
    nhid = w.shape[1]
    grid = (n // block_m,)
    return pl.pallas_call(
        functools.partial(_layer_body, nfeat=nfeat),
        grid=grid,
        in_specs=[
            pl.BlockSpec((block_m, n), lambda i: (i, 0)),
            pl.BlockSpec((n, nfeat), lambda i: (0, 0)),
            pl.BlockSpec((block_m, nfeat), lambda i: (i, 0)),
            pl.BlockSpec((2 * nfeat, nhid), lambda i: (0, 0)),
        ],
        out_specs=pl.BlockSpec((block_m, nhid), lambda i: (i, 0)),
        out_shape=jax.ShapeDtypeStruct((n, nhid), jnp.float32),
        compiler_params=pltpu.CompilerParams(
            dimension_semantics=("arbitrary",),
        ),
    )(adj, xin, xin, w)


@jax.jit
def kernel(x, adj, W1, W2):
    n = x.shape[0]
    block_m = next(
        (b for b in (512, 400, 256, 200, 128, 80, 8) if n % b == 0), n
    )
    x1 = _layer(x, adj, W1, block_m)
    return _layer(x1, adj, W2, block_m)
